# baseline (speedup 1.0000x reference)
import jax
import jax.numpy as jnp
from jax.experimental import pallas as pl
from jax.experimental.pallas import tpu as pltpu

B = 1024
D = 256
H = 256
K = 16384
BK = 2048
NSTEPS = K // BK
NA = 18
NA_PAD = 32


def _fused(q_ref, kT_ref, v_ref, h0_ref, c0_ref, wihT_ref, whhT_ref, b_ref,
           waT_ref, ba_ref, wc_ref, bc_ref,
           act_ref, val_ref, h_ref, c_ref,
           acc_ref, m_ref, d_ref):
    j = pl.program_id(0)

    @pl.when(j == 0)
    def _init():
        m_ref[...] = jnp.full_like(m_ref, -3.0e38)
        d_ref[...] = jnp.zeros_like(d_ref)
        acc_ref[...] = jnp.zeros_like(acc_ref)

    q = q_ref[...]
    kT = kT_ref[...]
    v = v_ref[...]

    kk = kT * kT
    k2 = jnp.sum(kk, axis=0, keepdims=True)
    s = jax.lax.dot_general(q, kT, (((1,), (0,)), ((), ())),
                            preferred_element_type=jnp.float32) - k2

    m_old = m_ref[...]
    m_new = jnp.maximum(m_old, jnp.max(s, axis=1, keepdims=True))
    alpha = jnp.exp(m_old - m_new)
    p = jnp.exp(s - m_new)
    d_new = d_ref[...] * alpha + jnp.sum(p, axis=1, keepdims=True)
    acc_new = acc_ref[...] * alpha + jax.lax.dot_general(
        p.astype(jnp.bfloat16), v.astype(jnp.bfloat16), (((1,), (0,)), ((), ())),
        preferred_element_type=jnp.float32)
    m_ref[...] = m_new
    d_ref[...] = d_new
    acc_ref[...] = acc_new

    @pl.when(j == NSTEPS - 1)
    def _epilogue():
        m_t = acc_new / d_new

        h0 = h0_ref[...]
        c0 = c0_ref[...]
        gates = (jax.lax.dot_general(q, wihT_ref[...], (((1,), (0,)), ((), ())),
                                     preferred_element_type=jnp.float32)
                 + jax.lax.dot_general(h0, whhT_ref[...], (((1,), (0,)), ((), ())),
                                       preferred_element_type=jnp.float32)
                 + b_ref[...])
        i_g = jax.nn.sigmoid(gates[:, 0:H])
        f_g = jax.nn.sigmoid(gates[:, H:2 * H])
        g_g = jnp.tanh(gates[:, 2 * H:3 * H])
        o_g = jax.nn.sigmoid(gates[:, 3 * H:4 * H])
        c_t = f_g * c0 + i_g * g_g + m_t
        h_t = o_g * jnp.tanh(c_t)
        c_ref[...] = c_t
        h_ref[...] = h_t

        logits = jax.lax.dot_general(h_t, waT_ref[...], (((1,), (0,)), ((), ())),
                                     preferred_element_type=jnp.float32) + ba_ref[...]
        col = jax.lax.broadcasted_iota(jnp.int32, (B, NA_PAD), 1)
        logits = jnp.where(col < NA, logits, -3.0e38)
        lmax = jnp.max(logits, axis=1, keepdims=True)
        e = jnp.exp(logits - lmax)
        act_ref[...] = e / jnp.sum(e, axis=1, keepdims=True)

        val_ref[...] = (jnp.sum(h_t * wc_ref[...], axis=1, keepdims=True)
                        + bc_ref[...])


def kernel(x_t, h0, c0, keys_mem, vals_mem, W_ih, W_hh, b_ih, b_hh,
           W_actor, b_actor, W_critic, b_critic):
    b, s_len, d = x_t.shape
    q = x_t.reshape(b, d) * 2.0
    kT = keys_mem.T
    wihT = W_ih.T * 0.5
    whhT = W_hh.T
    bias = (b_ih + b_hh).reshape(1, 4 * H)
    waT = jnp.zeros((H, NA_PAD), jnp.float32).at[:, :NA].set(W_actor.T)
    ba = jnp.zeros((1, NA_PAD), jnp.float32).at[0, :NA].set(b_actor)
    wc = W_critic.reshape(1, H)
    bc = b_critic.reshape(1, 1)

    full = lambda shp: pl.BlockSpec(shp, lambda j: (0, 0))
    out = pl.pallas_call(
        _fused,
        grid=(NSTEPS,),
        in_specs=[
            full((B, D)),
            pl.BlockSpec((D, BK), lambda j: (0, j)),
            pl.BlockSpec((BK, H), lambda j: (j, 0)),
            full((B, H)),
            full((B, H)),
            full((D, 4 * H)),
            full((H, 4 * H)),
            full((1, 4 * H)),
            full((H, NA_PAD)),
            full((1, NA_PAD)),
            full((1, H)),
            full((1, 1)),
        ],
        out_specs=[
            full((B, NA_PAD)),
            full((B, 1)),
            full((B, H)),
            full((B, H)),
        ],
        out_shape=[
            jax.ShapeDtypeStruct((B, NA_PAD), jnp.float32),
            jax.ShapeDtypeStruct((B, 1), jnp.float32),
            jax.ShapeDtypeStruct((B, H), jnp.float32),
            jax.ShapeDtypeStruct((B, H), jnp.float32),
        ],
        scratch_shapes=[
            pltpu.VMEM((B, H), jnp.float32),
            pltpu.VMEM((B, 1), jnp.float32),
            pltpu.VMEM((B, 1), jnp.float32),
        ],
        compiler_params=pltpu.CompilerParams(
            dimension_semantics=("arbitrary",),
        ),
    )(q, kT, vals_mem, h0[0], c0[0], wihT, whhT, bias, waT, ba, wc, bc)

    act_pad, val, h_t, c_t = out
    action_dist = act_pad[:, :NA].reshape(b, s_len, NA)
    value = val.reshape(b, s_len, 1)
    h_seq = h_t.reshape(b, s_len, H)
    c_out = c_t.reshape(1, b, H)
    return (action_dist, value, h_seq, c_out)


# speedup vs baseline: 1.2478x; 1.2478x over previous
import jax
import jax.numpy as jnp
from jax.experimental import pallas as pl
from jax.experimental.pallas import tpu as pltpu

B = 1024
D = 256
H = 256
K = 16384
BK = 2048
NSTEPS = K // BK
NA = 18
NA_PAD = 32


def _fused(q_ref, k_ref, k2_ref, v_ref, h0_ref, c0_ref, wih_ref, whh_ref, b_ref,
           wa_ref, ba_ref, wc_ref, bc_ref,
           act_ref, val_ref, h_ref, c_ref,
           acc_ref, m_ref, d_ref):
    j = pl.program_id(0)

    @pl.when(j == 0)
    def _init():
        m_ref[...] = jnp.full_like(m_ref, -3.0e38)
        d_ref[...] = jnp.zeros_like(d_ref)
        acc_ref[...] = jnp.zeros_like(acc_ref)

    q = q_ref[...]
    k = k_ref[...]
    v = v_ref[...]

    s = 2.0 * jax.lax.dot_general(q, k, (((1,), (1,)), ((), ())),
                                  preferred_element_type=jnp.float32) - k2_ref[...]

    m_old = m_ref[...]
    m_new = jnp.maximum(m_old, jnp.max(s, axis=1, keepdims=True))
    alpha = jnp.exp(m_old - m_new)
    p = jnp.exp(s - m_new)
    d_new = d_ref[...] * alpha + jnp.sum(p, axis=1, keepdims=True)
    acc_new = acc_ref[...] * alpha + jax.lax.dot_general(
        p, v, (((1,), (0,)), ((), ())),
        preferred_element_type=jnp.float32)
    m_ref[...] = m_new
    d_ref[...] = d_new
    acc_ref[...] = acc_new

    @pl.when(j == NSTEPS - 1)
    def _epilogue():
        m_t = acc_new / d_new

        h0 = h0_ref[...]
        c0 = c0_ref[...]
        gates = (jax.lax.dot_general(q, wih_ref[...], (((1,), (1,)), ((), ())),
                                     preferred_element_type=jnp.float32)
                 + jax.lax.dot_general(h0, whh_ref[...], (((1,), (1,)), ((), ())),
                                       preferred_element_type=jnp.float32)
                 + b_ref[...])
        i_g = jax.nn.sigmoid(gates[:, 0:H])
        f_g = jax.nn.sigmoid(gates[:, H:2 * H])
        g_g = jnp.tanh(gates[:, 2 * H:3 * H])
        o_g = jax.nn.sigmoid(gates[:, 3 * H:4 * H])
        c_t = f_g * c0 + i_g * g_g + m_t
        h_t = o_g * jnp.tanh(c_t)
        c_ref[...] = c_t
        h_ref[...] = h_t

        logits = jax.lax.dot_general(h_t, wa_ref[...], (((1,), (1,)), ((), ())),
                                     preferred_element_type=jnp.float32) + ba_ref[...]
        col = jax.lax.broadcasted_iota(jnp.int32, (B, NA_PAD), 1)
        logits = jnp.where(col < NA, logits, -3.0e38)
        lmax = jnp.max(logits, axis=1, keepdims=True)
        e = jnp.exp(logits - lmax)
        act_ref[...] = e / jnp.sum(e, axis=1, keepdims=True)

        val_ref[...] = (jnp.sum(h_t * wc_ref[...], axis=1, keepdims=True)
                        + bc_ref[...])


def kernel(x_t, h0, c0, keys_mem, vals_mem, W_ih, W_hh, b_ih, b_hh,
           W_actor, b_actor, W_critic, b_critic):
    b, s_len, d = x_t.shape
    q = x_t.reshape(b, d)
    k2 = jnp.sum(keys_mem * keys_mem, axis=1).reshape(1, K)
    bias = (b_ih + b_hh).reshape(1, 4 * H)
    wa = jnp.zeros((NA_PAD, H), jnp.float32).at[:NA, :].set(W_actor)
    ba = jnp.zeros((1, NA_PAD), jnp.float32).at[0, :NA].set(b_actor)
    wc = W_critic.reshape(1, H)
    bc = b_critic.reshape(1, 1)

    full = lambda shp: pl.BlockSpec(shp, lambda j: (0, 0))
    out = pl.pallas_call(
        _fused,
        grid=(NSTEPS,),
        in_specs=[
            full((B, D)),
            pl.BlockSpec((BK, D), lambda j: (j, 0)),
            pl.BlockSpec((1, BK), lambda j: (0, j)),
            pl.BlockSpec((BK, H), lambda j: (j, 0)),
            full((B, H)),
            full((B, H)),
            full((4 * H, D)),
            full((4 * H, H)),
            full((1, 4 * H)),
            full((NA_PAD, H)),
            full((1, NA_PAD)),
            full((1, H)),
            full((1, 1)),
        ],
        out_specs=[
            full((B, NA_PAD)),
            full((B, 1)),
            full((B, H)),
            full((B, H)),
        ],
        out_shape=[
            jax.ShapeDtypeStruct((B, NA_PAD), jnp.float32),
            jax.ShapeDtypeStruct((B, 1), jnp.float32),
            jax.ShapeDtypeStruct((B, H), jnp.float32),
            jax.ShapeDtypeStruct((B, H), jnp.float32),
        ],
        scratch_shapes=[
            pltpu.VMEM((B, H), jnp.float32),
            pltpu.VMEM((B, 1), jnp.float32),
            pltpu.VMEM((B, 1), jnp.float32),
        ],
        compiler_params=pltpu.CompilerParams(
            dimension_semantics=("arbitrary",),
        ),
    )(q, keys_mem, k2, vals_mem, h0[0], c0[0], W_ih, W_hh, bias, wa, ba, wc, bc)

    act_pad, val, h_t, c_t = out
    action_dist = act_pad[:, :NA].reshape(b, s_len, NA)
    value = val.reshape(b, s_len, 1)
    h_seq = h_t.reshape(b, s_len, H)
    c_out = c_t.reshape(1, b, H)
    return (action_dist, value, h_seq, c_out)
